# baseline (speedup 1.0000x reference)
import functools

import jax
import jax.numpy as jnp
from jax import lax
from jax.experimental import pallas as pl
from jax.experimental.pallas import tpu as pltpu
from jax.experimental.pallas import tpu_sc as plsc

N_NODES = 10000
N_EDGES = 320000
N_GRAPHS = 32
D = 128
D_OUT = 64

NC = 2
NS = 16
NW = NC * NS

N_PAD = 10112
ROWS_PER_TILE = N_PAD // NS
CHUNK = 128
E_PAD = 327680
CHUNKS_PER_TILE = E_PAD // (NW * CHUNK)

_mesh = plsc.VectorSubcoreMesh(core_axis_name="c", subcore_axis_name="s",
                               num_cores=NC, num_subcores=NS)


KB = 8
GROUPS = CHUNKS_PER_TILE // KB


def _sc_agg_body(h_hbm, src_hbm, dst_hbm, zrow_hbm, out_hbm,
                 srcv, dstv, rows, acc):
    c = lax.axis_index("c")
    s = lax.axis_index("s")
    wid = c * NS + s
    row0 = s * ROWS_PER_TILE
    pltpu.sync_copy(zrow_hbm, acc.at[pl.ds(row0, ROWS_PER_TILE)])
    plsc.subcore_barrier()

    def step(g, carry):
        pltpu.sync_copy(src_hbm.at[wid, pl.ds(g * KB, KB)], srcv)
        pltpu.sync_copy(dst_hbm.at[wid, pl.ds(g * KB, KB)], dstv)
        for b in range(KB):
            pltpu.sync_copy(h_hbm.at[srcv.at[b]], rows)
            pltpu.sync_copy(rows, acc.at[dstv.at[b]], add=True)
        return carry

    lax.fori_loop(0, GROUPS, step, 0)
    plsc.subcore_barrier()
    pltpu.sync_copy(acc.at[pl.ds(row0, ROWS_PER_TILE)],
                    out_hbm.at[c, pl.ds(row0, ROWS_PER_TILE)])


_sc_agg = pl.kernel(
    _sc_agg_body,
    out_type=jax.ShapeDtypeStruct((NC, N_PAD, D), jnp.float32),
    mesh=_mesh,
    scratch_types=[
        pltpu.VMEM((KB, CHUNK), jnp.int32),
        pltpu.VMEM((KB, CHUNK), jnp.int32),
        pltpu.VMEM((CHUNK, D), jnp.float32),
        pltpu.VMEM_SHARED((N_PAD, D), jnp.float32),
    ],
)


R = 1000
GRID = N_NODES // R


def _tc_layer_body(relu, acc_ref, cnt_ref, h_ref, wr_ref, b_ref, wo_ref, o_ref):
    ssum = acc_ref[0] + acc_ref[1]
    cnt = cnt_ref[0][:, 0:1] + cnt_ref[1][:, 0:1]
    mean = ssum / jnp.maximum(cnt, 1.0)
    y = (jnp.dot(mean, wr_ref[...], preferred_element_type=jnp.float32)
         + jnp.dot(h_ref[...], wo_ref[...], preferred_element_type=jnp.float32)
         + b_ref[...])
    o_ref[...] = jnp.maximum(y, 0.0) if relu else y


def _make_tc_layer(relu):
    return pl.pallas_call(
        functools.partial(_tc_layer_body, relu),
        grid=(GRID,),
        in_specs=[
            pl.BlockSpec((2, R, D), lambda i: (0, i, 0)),
            pl.BlockSpec((2, R, D), lambda i: (0, i, 0)),
            pl.BlockSpec((R, D), lambda i: (i, 0)),
            pl.BlockSpec((D, D), lambda i: (0, 0)),
            pl.BlockSpec((1, D), lambda i: (0, 0)),
            pl.BlockSpec((D, D), lambda i: (0, 0)),
        ],
        out_specs=pl.BlockSpec((R, D), lambda i: (i, 0)),
        out_shape=jax.ShapeDtypeStruct((N_NODES, D), jnp.float32),
    )


_tc_layer_relu = _make_tc_layer(True)


def _tc_pool_body(acc_ref, cnt_ref, h_ref, wr_ref, b_ref, wo_ref, batch_ref,
                  wl_ref, bl_ref, o_ref, pooled, gcnt):
    i = pl.program_id(0)

    @pl.when(i == 0)
    def _():
        pooled[...] = jnp.zeros_like(pooled)
        gcnt[...] = jnp.zeros_like(gcnt)

    ssum = acc_ref[0] + acc_ref[1]
    cnt = cnt_ref[0][:, 0:1] + cnt_ref[1][:, 0:1]
    mean = ssum / jnp.maximum(cnt, 1.0)
    h3 = (jnp.dot(mean, wr_ref[...], preferred_element_type=jnp.float32)
          + jnp.dot(h_ref[...], wo_ref[...], preferred_element_type=jnp.float32)
          + b_ref[...])
    onehot = (batch_ref[...] ==
              lax.broadcasted_iota(jnp.int32, (R, N_GRAPHS), 1)).astype(jnp.float32)
    pooled[...] += lax.dot_general(onehot, h3, (((0,), (0,)), ((), ())),
                                   preferred_element_type=jnp.float32)
    gcnt[...] += jnp.broadcast_to(jnp.sum(onehot, axis=0)[:, None], (N_GRAPHS, D))

    @pl.when(i == GRID - 1)
    def _():
        pm = pooled[...] / jnp.maximum(gcnt[...], 1.0)
        o_ref[...] = (jnp.dot(pm, wl_ref[...], preferred_element_type=jnp.float32)
                      + bl_ref[...])


_tc_pool = pl.pallas_call(
    _tc_pool_body,
    grid=(GRID,),
    in_specs=[
        pl.BlockSpec((2, R, D), lambda i: (0, i, 0)),
        pl.BlockSpec((2, R, D), lambda i: (0, i, 0)),
        pl.BlockSpec((R, D), lambda i: (i, 0)),
        pl.BlockSpec((D, D), lambda i: (0, 0)),
        pl.BlockSpec((1, D), lambda i: (0, 0)),
        pl.BlockSpec((D, D), lambda i: (0, 0)),
        pl.BlockSpec((R, 1), lambda i: (i, 0)),
        pl.BlockSpec((D, D_OUT), lambda i: (0, 0)),
        pl.BlockSpec((1, D_OUT), lambda i: (0, 0)),
    ],
    out_specs=pl.BlockSpec((N_GRAPHS, D_OUT), lambda i: (0, 0)),
    out_shape=jax.ShapeDtypeStruct((N_GRAPHS, D_OUT), jnp.float32),
    scratch_shapes=[
        pltpu.VMEM((N_GRAPHS, D), jnp.float32),
        pltpu.VMEM((N_GRAPHS, D), jnp.float32),
    ],
)


def kernel(x, edge_index, batch,
           W_rel1, b_rel1, W_root1,
           W_rel2, b_rel2, W_root2,
           W_rel3, b_rel3, W_root3,
           W_lin, b_lin):
    src = edge_index[0].astype(jnp.int32)
    dst = edge_index[1].astype(jnp.int32)
    pad_e = E_PAD - N_EDGES
    src_p = jnp.concatenate(
        [src, jnp.zeros((pad_e,), jnp.int32)]).reshape(NW, CHUNKS_PER_TILE, CHUNK)
    dst_p = jnp.concatenate(
        [dst, jnp.full((pad_e,), N_NODES, jnp.int32)]).reshape(NW, CHUNKS_PER_TILE, CHUNK)
    zrow = jnp.zeros((ROWS_PER_TILE, D), jnp.float32)
    ones_tab = jnp.ones((8, D), jnp.float32)
    src0_p = jnp.zeros((NW, CHUNKS_PER_TILE, CHUNK), jnp.int32)
    batch2d = batch.astype(jnp.int32).reshape(N_NODES, 1)

    wr1t, wo1t = W_rel1.T, W_root1.T
    wr2t, wo2t = W_rel2.T, W_root2.T
    wr3t, wo3t = W_rel3.T, W_root3.T
    wlt = W_lin.T
    b1 = b_rel1.reshape(1, D)
    b2 = b_rel2.reshape(1, D)
    b3 = b_rel3.reshape(1, D)
    bl = b_lin.reshape(1, D_OUT)

    cnt = _sc_agg(ones_tab, src0_p, dst_p, zrow)
    acc1 = _sc_agg(x, src_p, dst_p, zrow)
    h1 = _tc_layer_relu(acc1, cnt, x, wr1t, b1, wo1t)
    acc2 = _sc_agg(h1, src_p, dst_p, zrow)
    h2 = _tc_layer_relu(acc2, cnt, h1, wr2t, b2, wo2t)
    acc3 = _sc_agg(h2, src_p, dst_p, zrow)
    return _tc_pool(acc3, cnt, h2, wr3t, b3, wo3t, batch2d, wlt, bl)


# speedup vs baseline: 6.2233x; 6.2233x over previous
import functools

import jax
import jax.numpy as jnp
from jax import lax
from jax.experimental import pallas as pl
from jax.experimental.pallas import tpu as pltpu
from jax.experimental.pallas import tpu_sc as plsc

N_NODES = 10000
N_EDGES = 320000
N_GRAPHS = 32
D = 128
D_OUT = 64

NC = 2
NS = 16
NW = NC * NS

N_PAD = 10112
ROWS_PER_TILE = N_PAD // NS
CHUNK = 128
E_PAD = 327680
CHUNKS_PER_TILE = E_PAD // (NW * CHUNK)

_mesh = plsc.VectorSubcoreMesh(core_axis_name="c", subcore_axis_name="s",
                               num_cores=NC, num_subcores=NS)


KB = 8
GROUPS = CHUNKS_PER_TILE // KB


def _sc_agg_body(h_hbm, src_hbm, dst_hbm, zrow_hbm, out_hbm,
                 srcv, dstv, rows, acc):
    c = lax.axis_index("c")
    s = lax.axis_index("s")
    wid = c * NS + s
    row0 = s * ROWS_PER_TILE
    pltpu.sync_copy(zrow_hbm, acc.at[pl.ds(row0, ROWS_PER_TILE)])
    plsc.subcore_barrier()

    def step(g, carry):
        pltpu.sync_copy(src_hbm.at[wid, pl.ds(g * KB, KB)], srcv)
        pltpu.sync_copy(dst_hbm.at[wid, pl.ds(g * KB, KB)], dstv)
        for b in range(KB):
            pltpu.sync_copy(h_hbm.at[srcv.at[b]], rows)
            pltpu.sync_copy(rows, acc.at[dstv.at[b]], add=True)
        return carry

    lax.fori_loop(0, GROUPS, step, 0)
    plsc.subcore_barrier()
    pltpu.sync_copy(acc.at[pl.ds(row0, ROWS_PER_TILE)],
                    out_hbm.at[c, pl.ds(row0, ROWS_PER_TILE)])


_sc_agg = pl.kernel(
    _sc_agg_body,
    out_type=jax.ShapeDtypeStruct((NC, N_PAD, D), jnp.float32),
    mesh=_mesh,
    scratch_types=[
        pltpu.VMEM((KB, CHUNK), jnp.int32),
        pltpu.VMEM((KB, CHUNK), jnp.int32),
        pltpu.VMEM((CHUNK, D), jnp.float32),
        pltpu.VMEM_SHARED((N_PAD, D), jnp.float32),
    ],
)


R = 1000
GRID = N_NODES // R


def _tc_layer_body(relu, acc_ref, cnt_ref, h_ref, wr_ref, b_ref, wo_ref, o_ref):
    ssum = acc_ref[0] + acc_ref[1]
    cnt = cnt_ref[0][:, 0:1] + cnt_ref[1][:, 0:1]
    mean = ssum / jnp.maximum(cnt, 1.0)
    y = (jnp.dot(mean, wr_ref[...], preferred_element_type=jnp.float32)
         + jnp.dot(h_ref[...], wo_ref[...], preferred_element_type=jnp.float32)
         + b_ref[...])
    o_ref[...] = jnp.maximum(y, 0.0) if relu else y


def _make_tc_layer(relu):
    return pl.pallas_call(
        functools.partial(_tc_layer_body, relu),
        grid=(GRID,),
        in_specs=[
            pl.BlockSpec((2, R, D), lambda i: (0, i, 0)),
            pl.BlockSpec((2, R, D), lambda i: (0, i, 0)),
            pl.BlockSpec((R, D), lambda i: (i, 0)),
            pl.BlockSpec((D, D), lambda i: (0, 0)),
            pl.BlockSpec((1, D), lambda i: (0, 0)),
            pl.BlockSpec((D, D), lambda i: (0, 0)),
        ],
        out_specs=pl.BlockSpec((R, D), lambda i: (i, 0)),
        out_shape=jax.ShapeDtypeStruct((N_NODES, D), jnp.float32),
    )


_tc_layer_relu = _make_tc_layer(True)


def _tc_pool_body(acc_ref, cnt_ref, h_ref, wr_ref, b_ref, wo_ref, batch_ref,
                  wl_ref, bl_ref, o_ref, pooled, gcnt):
    i = pl.program_id(0)

    @pl.when(i == 0)
    def _():
        pooled[...] = jnp.zeros_like(pooled)
        gcnt[...] = jnp.zeros_like(gcnt)

    ssum = acc_ref[0] + acc_ref[1]
    cnt = cnt_ref[0][:, 0:1] + cnt_ref[1][:, 0:1]
    mean = ssum / jnp.maximum(cnt, 1.0)
    h3 = (jnp.dot(mean, wr_ref[...], preferred_element_type=jnp.float32)
          + jnp.dot(h_ref[...], wo_ref[...], preferred_element_type=jnp.float32)
          + b_ref[...])
    onehot = (batch_ref[...] ==
              lax.broadcasted_iota(jnp.int32, (R, N_GRAPHS), 1)).astype(jnp.float32)
    pooled[...] += lax.dot_general(onehot, h3, (((0,), (0,)), ((), ())),
                                   preferred_element_type=jnp.float32)
    gcnt[...] += jnp.broadcast_to(jnp.sum(onehot, axis=0)[:, None], (N_GRAPHS, D))

    @pl.when(i == GRID - 1)
    def _():
        pm = pooled[...] / jnp.maximum(gcnt[...], 1.0)
        o_ref[...] = (jnp.dot(pm, wl_ref[...], preferred_element_type=jnp.float32)
                      + bl_ref[...])


_tc_pool = pl.pallas_call(
    _tc_pool_body,
    grid=(GRID,),
    in_specs=[
        pl.BlockSpec((2, R, D), lambda i: (0, i, 0)),
        pl.BlockSpec((2, R, D), lambda i: (0, i, 0)),
        pl.BlockSpec((R, D), lambda i: (i, 0)),
        pl.BlockSpec((D, D), lambda i: (0, 0)),
        pl.BlockSpec((1, D), lambda i: (0, 0)),
        pl.BlockSpec((D, D), lambda i: (0, 0)),
        pl.BlockSpec((R, 1), lambda i: (i, 0)),
        pl.BlockSpec((D, D_OUT), lambda i: (0, 0)),
        pl.BlockSpec((1, D_OUT), lambda i: (0, 0)),
    ],
    out_specs=pl.BlockSpec((N_GRAPHS, D_OUT), lambda i: (0, 0)),
    out_shape=jax.ShapeDtypeStruct((N_GRAPHS, D_OUT), jnp.float32),
    scratch_shapes=[
        pltpu.VMEM((N_GRAPHS, D), jnp.float32),
        pltpu.VMEM((N_GRAPHS, D), jnp.float32),
    ],
)


def kernel(x, edge_index, batch,
           W_rel1, b_rel1, W_root1,
           W_rel2, b_rel2, W_root2,
           W_rel3, b_rel3, W_root3,
           W_lin, b_lin):
    src = edge_index[0].astype(jnp.int32)
    dst = edge_index[1].astype(jnp.int32)
    pad_e = E_PAD - N_EDGES
    src_p = jnp.concatenate(
        [src, jnp.zeros((pad_e,), jnp.int32)]).reshape(NW, CHUNKS_PER_TILE, CHUNK)
    dst_p = jnp.concatenate(
        [dst, jnp.full((pad_e,), N_NODES, jnp.int32)]).reshape(NW, CHUNKS_PER_TILE, CHUNK)
    zrow = jnp.zeros((ROWS_PER_TILE, D), jnp.float32)
    ones_tab = jnp.ones((N_NODES, D), jnp.float32)
    batch2d = batch.astype(jnp.int32).reshape(N_NODES, 1)

    wr1t, wo1t = W_rel1.T, W_root1.T
    wr2t, wo2t = W_rel2.T, W_root2.T
    wr3t, wo3t = W_rel3.T, W_root3.T
    wlt = W_lin.T
    b1 = b_rel1.reshape(1, D)
    b2 = b_rel2.reshape(1, D)
    b3 = b_rel3.reshape(1, D)
    bl = b_lin.reshape(1, D_OUT)

    cnt = _sc_agg(ones_tab, src_p, dst_p, zrow)
    acc1 = _sc_agg(x, src_p, dst_p, zrow)
    h1 = _tc_layer_relu(acc1, cnt, x, wr1t, b1, wo1t)
    acc2 = _sc_agg(h1, src_p, dst_p, zrow)
    h2 = _tc_layer_relu(acc2, cnt, h1, wr2t, b2, wo2t)
    acc3 = _sc_agg(h2, src_p, dst_p, zrow)
    return _tc_pool(acc3, cnt, h2, wr3t, b3, wo3t, batch2d, wlt, bl)


# speedup vs baseline: 16.8654x; 2.7100x over previous
import functools

import jax
import jax.numpy as jnp
from jax import lax
from jax.experimental import pallas as pl
from jax.experimental.pallas import tpu as pltpu
from jax.experimental.pallas import tpu_sc as plsc

N_NODES = 10000
N_EDGES = 320000
N_GRAPHS = 32
D = 128
D_OUT = 64

NC = 2
NS = 16
NW = NC * NS

N_PAD = 10112
ROWS_PER_TILE = N_PAD // NS
CHUNK = 128
E_PAD = 327680
CHUNKS_PER_TILE = E_PAD // (NW * CHUNK)

_mesh = plsc.VectorSubcoreMesh(core_axis_name="c", subcore_axis_name="s",
                               num_cores=NC, num_subcores=NS)


KB = 8
GROUPS = CHUNKS_PER_TILE // KB


def _sc_agg_body(h_hbm, src_hbm, dst_hbm, zrow_hbm, out_hbm,
                 srcv, dstv, rows, acc):
    c = lax.axis_index("c")
    s = lax.axis_index("s")
    wid = c * NS + s
    row0 = s * ROWS_PER_TILE
    pltpu.sync_copy(zrow_hbm, acc.at[pl.ds(row0, ROWS_PER_TILE)])
    plsc.subcore_barrier()

    def step(g, carry):
        pltpu.sync_copy(src_hbm.at[wid, pl.ds(g * KB, KB)], srcv)
        pltpu.sync_copy(dst_hbm.at[wid, pl.ds(g * KB, KB)], dstv)
        for b in range(KB):
            pltpu.sync_copy(h_hbm.at[srcv.at[b]], rows)
            pltpu.sync_copy(rows, acc.at[dstv.at[b]], add=True)
        return carry

    lax.fori_loop(0, GROUPS, step, 0)
    plsc.subcore_barrier()
    pltpu.sync_copy(acc.at[pl.ds(row0, ROWS_PER_TILE)],
                    out_hbm.at[c, pl.ds(row0, ROWS_PER_TILE)])


_sc_agg = pl.kernel(
    _sc_agg_body,
    out_type=jax.ShapeDtypeStruct((NC, N_PAD, D), jnp.float32),
    mesh=_mesh,
    scratch_types=[
        pltpu.VMEM((KB, CHUNK), jnp.int32),
        pltpu.VMEM((KB, CHUNK), jnp.int32),
        pltpu.VMEM((CHUNK, D), jnp.float32),
        pltpu.VMEM_SHARED((N_PAD, D), jnp.float32),
    ],
)


R = 1000
GRID = N_NODES // R


def _tc_layer_body(relu, acc_ref, cnt_ref, h_ref, wr_ref, b_ref, wo_ref, o_ref):
    ssum = acc_ref[0] + acc_ref[1]
    cnt = cnt_ref[0][:, 0:1] + cnt_ref[1][:, 0:1]
    mean = ssum / jnp.maximum(cnt, 1.0)
    y = (jnp.dot(mean, wr_ref[...], preferred_element_type=jnp.float32)
         + jnp.dot(h_ref[...], wo_ref[...], preferred_element_type=jnp.float32)
         + b_ref[...])
    o_ref[...] = jnp.maximum(y, 0.0) if relu else y


def _make_tc_layer(relu):
    return pl.pallas_call(
        functools.partial(_tc_layer_body, relu),
        grid=(GRID,),
        in_specs=[
            pl.BlockSpec((2, R, D), lambda i: (0, i, 0)),
            pl.BlockSpec((2, R, D), lambda i: (0, i, 0)),
            pl.BlockSpec((R, D), lambda i: (i, 0)),
            pl.BlockSpec((D, D), lambda i: (0, 0)),
            pl.BlockSpec((1, D), lambda i: (0, 0)),
            pl.BlockSpec((D, D), lambda i: (0, 0)),
        ],
        out_specs=pl.BlockSpec((R, D), lambda i: (i, 0)),
        out_shape=jax.ShapeDtypeStruct((N_NODES, D), jnp.float32),
    )


_tc_layer_relu = _make_tc_layer(True)


def _tc_pool_body(acc_ref, cnt_ref, h_ref, wr_ref, b_ref, wo_ref, batch_ref,
                  wl_ref, bl_ref, o_ref, pooled, gcnt):
    i = pl.program_id(0)

    @pl.when(i == 0)
    def _():
        pooled[...] = jnp.zeros_like(pooled)
        gcnt[...] = jnp.zeros_like(gcnt)

    ssum = acc_ref[0] + acc_ref[1]
    cnt = cnt_ref[0][:, 0:1] + cnt_ref[1][:, 0:1]
    mean = ssum / jnp.maximum(cnt, 1.0)
    h3 = (jnp.dot(mean, wr_ref[...], preferred_element_type=jnp.float32)
          + jnp.dot(h_ref[...], wo_ref[...], preferred_element_type=jnp.float32)
          + b_ref[...])
    onehot = (batch_ref[...] ==
              lax.broadcasted_iota(jnp.int32, (R, N_GRAPHS), 1)).astype(jnp.float32)
    pooled[...] += lax.dot_general(onehot, h3, (((0,), (0,)), ((), ())),
                                   preferred_element_type=jnp.float32)
    gcnt[...] += jnp.broadcast_to(jnp.sum(onehot, axis=0)[:, None], (N_GRAPHS, D))

    @pl.when(i == GRID - 1)
    def _():
        pm = pooled[...] / jnp.maximum(gcnt[...], 1.0)
        o_ref[...] = (jnp.dot(pm, wl_ref[...], preferred_element_type=jnp.float32)
                      + bl_ref[...])


_tc_pool = pl.pallas_call(
    _tc_pool_body,
    grid=(GRID,),
    in_specs=[
        pl.BlockSpec((2, R, D), lambda i: (0, i, 0)),
        pl.BlockSpec((2, R, D), lambda i: (0, i, 0)),
        pl.BlockSpec((R, D), lambda i: (i, 0)),
        pl.BlockSpec((D, D), lambda i: (0, 0)),
        pl.BlockSpec((1, D), lambda i: (0, 0)),
        pl.BlockSpec((D, D), lambda i: (0, 0)),
        pl.BlockSpec((R, 1), lambda i: (i, 0)),
        pl.BlockSpec((D, D_OUT), lambda i: (0, 0)),
        pl.BlockSpec((1, D_OUT), lambda i: (0, 0)),
    ],
    out_specs=pl.BlockSpec((N_GRAPHS, D_OUT), lambda i: (0, 0)),
    out_shape=jax.ShapeDtypeStruct((N_GRAPHS, D_OUT), jnp.float32),
    scratch_shapes=[
        pltpu.VMEM((N_GRAPHS, D), jnp.float32),
        pltpu.VMEM((N_GRAPHS, D), jnp.float32),
    ],
)


def kernel(x, edge_index, batch,
           W_rel1, b_rel1, W_root1,
           W_rel2, b_rel2, W_root2,
           W_rel3, b_rel3, W_root3,
           W_lin, b_lin):
    src = edge_index[0].astype(jnp.int32)
    dst = edge_index[1].astype(jnp.int32)
    pad_e = E_PAD - N_EDGES
    pad_src = jnp.arange(pad_e, dtype=jnp.int32) % N_NODES
    pad_dst = N_NODES + (jnp.arange(pad_e, dtype=jnp.int32) % (N_PAD - N_NODES))
    src_p = jnp.concatenate([src, pad_src]).reshape(NW, CHUNKS_PER_TILE, CHUNK)
    dst_p = jnp.concatenate([dst, pad_dst]).reshape(NW, CHUNKS_PER_TILE, CHUNK)
    zrow = jnp.zeros((ROWS_PER_TILE, D), jnp.float32)
    ones_tab = jnp.ones((N_NODES, D), jnp.float32)
    batch2d = batch.astype(jnp.int32).reshape(N_NODES, 1)

    wr1t, wo1t = W_rel1.T, W_root1.T
    wr2t, wo2t = W_rel2.T, W_root2.T
    wr3t, wo3t = W_rel3.T, W_root3.T
    wlt = W_lin.T
    b1 = b_rel1.reshape(1, D)
    b2 = b_rel2.reshape(1, D)
    b3 = b_rel3.reshape(1, D)
    bl = b_lin.reshape(1, D_OUT)

    cnt = _sc_agg(ones_tab, src_p, dst_p, zrow)
    acc1 = _sc_agg(x, src_p, dst_p, zrow)
    h1 = _tc_layer_relu(acc1, cnt, x, wr1t, b1, wo1t)
    acc2 = _sc_agg(h1, src_p, dst_p, zrow)
    h2 = _tc_layer_relu(acc2, cnt, h1, wr2t, b2, wo2t)
    acc3 = _sc_agg(h2, src_p, dst_p, zrow)
    return _tc_pool(acc3, cnt, h2, wr3t, b3, wo3t, batch2d, wlt, bl)


# speedup vs baseline: 25.5277x; 1.5136x over previous
import functools

import jax
import jax.numpy as jnp
from jax import lax
from jax.experimental import pallas as pl
from jax.experimental.pallas import tpu as pltpu
from jax.experimental.pallas import tpu_sc as plsc

N_NODES = 10000
N_EDGES = 320000
N_GRAPHS = 32
D = 128
D_OUT = 64

NC = 2
NS = 16
NW = NC * NS

N_PAD = 10112
ROWS_PER_TILE = N_PAD // NS
CHUNK = 128
E_PAD = 327680
CHUNKS_PER_TILE = E_PAD // (NW * CHUNK)

_mesh = plsc.VectorSubcoreMesh(core_axis_name="c", subcore_axis_name="s",
                               num_cores=NC, num_subcores=NS)


KB = 8
GROUPS = CHUNKS_PER_TILE // KB


def _sc_agg_body(h_hbm, src_hbm, dst_hbm, zrow_hbm, out_hbm,
                 srcA, dstA, srcB, dstB, rows0, rows1, acc, semg, semi):
    c = lax.axis_index("c")
    s = lax.axis_index("s")
    wid = c * NS + s
    row0 = s * ROWS_PER_TILE
    pltpu.sync_copy(zrow_hbm, acc.at[pl.ds(row0, ROWS_PER_TILE)])
    pltpu.sync_copy(src_hbm.at[wid, pl.ds(0, KB)], srcA)
    pltpu.sync_copy(dst_hbm.at[wid, pl.ds(0, KB)], dstA)
    plsc.subcore_barrier()

    rows = (rows0, rows1)

    def process(srcv, dstv):
        pltpu.async_copy(h_hbm.at[srcv.at[0]], rows[0], semg)
        for b in range(KB):
            if b + 1 < KB:
                pltpu.async_copy(h_hbm.at[srcv.at[b + 1]], rows[(b + 1) % 2], semg)
            pltpu.make_async_copy(h_hbm.at[srcv.at[b]], rows[b % 2], semg).wait()
            pltpu.sync_copy(rows[b % 2], acc.at[dstv.at[b]], add=True)

    def stage(off, sv, dv):
        pltpu.async_copy(src_hbm.at[wid, pl.ds(off, KB)], sv, semi)
        pltpu.async_copy(dst_hbm.at[wid, pl.ds(off, KB)], dv, semi)

    def drain(off, sv, dv):
        pltpu.make_async_copy(src_hbm.at[wid, pl.ds(off, KB)], sv, semi).wait()
        pltpu.make_async_copy(dst_hbm.at[wid, pl.ds(off, KB)], dv, semi).wait()

    def step(i, carry):
        off1 = (2 * i + 1) * KB
        stage(off1, srcB, dstB)
        process(srcA, dstA)
        drain(off1, srcB, dstB)
        off2 = jnp.minimum(2 * i + 2, GROUPS - 1) * KB
        stage(off2, srcA, dstA)
        process(srcB, dstB)
        drain(off2, srcA, dstA)
        return carry

    lax.fori_loop(0, GROUPS // 2, step, 0)
    plsc.subcore_barrier()
    pltpu.sync_copy(acc.at[pl.ds(row0, ROWS_PER_TILE)],
                    out_hbm.at[c, pl.ds(row0, ROWS_PER_TILE)])


_sc_agg = pl.kernel(
    _sc_agg_body,
    out_type=jax.ShapeDtypeStruct((NC, N_PAD, D), jnp.float32),
    mesh=_mesh,
    scratch_types=[
        pltpu.VMEM((KB, CHUNK), jnp.int32),
        pltpu.VMEM((KB, CHUNK), jnp.int32),
        pltpu.VMEM((KB, CHUNK), jnp.int32),
        pltpu.VMEM((KB, CHUNK), jnp.int32),
        pltpu.VMEM((CHUNK, D), jnp.float32),
        pltpu.VMEM((CHUNK, D), jnp.float32),
        pltpu.VMEM_SHARED((N_PAD, D), jnp.float32),
        pltpu.SemaphoreType.DMA,
        pltpu.SemaphoreType.DMA,
    ],
)


R = 1000
GRID = N_NODES // R


def _tc_layer_body(relu, acc_ref, cnt_ref, h_ref, wr_ref, b_ref, wo_ref, o_ref):
    ssum = acc_ref[0] + acc_ref[1]
    cnt = cnt_ref[0][:, 0:1] + cnt_ref[1][:, 0:1]
    mean = ssum / jnp.maximum(cnt, 1.0)
    y = (jnp.dot(mean, wr_ref[...], preferred_element_type=jnp.float32)
         + jnp.dot(h_ref[...], wo_ref[...], preferred_element_type=jnp.float32)
         + b_ref[...])
    o_ref[...] = jnp.maximum(y, 0.0) if relu else y


def _make_tc_layer(relu):
    return pl.pallas_call(
        functools.partial(_tc_layer_body, relu),
        grid=(GRID,),
        in_specs=[
            pl.BlockSpec((2, R, D), lambda i: (0, i, 0)),
            pl.BlockSpec((2, R, D), lambda i: (0, i, 0)),
            pl.BlockSpec((R, D), lambda i: (i, 0)),
            pl.BlockSpec((D, D), lambda i: (0, 0)),
            pl.BlockSpec((1, D), lambda i: (0, 0)),
            pl.BlockSpec((D, D), lambda i: (0, 0)),
        ],
        out_specs=pl.BlockSpec((R, D), lambda i: (i, 0)),
        out_shape=jax.ShapeDtypeStruct((N_NODES, D), jnp.float32),
    )


_tc_layer_relu = _make_tc_layer(True)


def _tc_pool_body(acc_ref, cnt_ref, h_ref, wr_ref, b_ref, wo_ref, batch_ref,
                  wl_ref, bl_ref, o_ref, pooled, gcnt):
    i = pl.program_id(0)

    @pl.when(i == 0)
    def _():
        pooled[...] = jnp.zeros_like(pooled)
        gcnt[...] = jnp.zeros_like(gcnt)

    ssum = acc_ref[0] + acc_ref[1]
    cnt = cnt_ref[0][:, 0:1] + cnt_ref[1][:, 0:1]
    mean = ssum / jnp.maximum(cnt, 1.0)
    h3 = (jnp.dot(mean, wr_ref[...], preferred_element_type=jnp.float32)
          + jnp.dot(h_ref[...], wo_ref[...], preferred_element_type=jnp.float32)
          + b_ref[...])
    onehot = (batch_ref[...] ==
              lax.broadcasted_iota(jnp.int32, (R, N_GRAPHS), 1)).astype(jnp.float32)
    pooled[...] += lax.dot_general(onehot, h3, (((0,), (0,)), ((), ())),
                                   preferred_element_type=jnp.float32)
    gcnt[...] += jnp.broadcast_to(jnp.sum(onehot, axis=0)[:, None], (N_GRAPHS, D))

    @pl.when(i == GRID - 1)
    def _():
        pm = pooled[...] / jnp.maximum(gcnt[...], 1.0)
        o_ref[...] = (jnp.dot(pm, wl_ref[...], preferred_element_type=jnp.float32)
                      + bl_ref[...])


_tc_pool = pl.pallas_call(
    _tc_pool_body,
    grid=(GRID,),
    in_specs=[
        pl.BlockSpec((2, R, D), lambda i: (0, i, 0)),
        pl.BlockSpec((2, R, D), lambda i: (0, i, 0)),
        pl.BlockSpec((R, D), lambda i: (i, 0)),
        pl.BlockSpec((D, D), lambda i: (0, 0)),
        pl.BlockSpec((1, D), lambda i: (0, 0)),
        pl.BlockSpec((D, D), lambda i: (0, 0)),
        pl.BlockSpec((R, 1), lambda i: (i, 0)),
        pl.BlockSpec((D, D_OUT), lambda i: (0, 0)),
        pl.BlockSpec((1, D_OUT), lambda i: (0, 0)),
    ],
    out_specs=pl.BlockSpec((N_GRAPHS, D_OUT), lambda i: (0, 0)),
    out_shape=jax.ShapeDtypeStruct((N_GRAPHS, D_OUT), jnp.float32),
    scratch_shapes=[
        pltpu.VMEM((N_GRAPHS, D), jnp.float32),
        pltpu.VMEM((N_GRAPHS, D), jnp.float32),
    ],
)


def kernel(x, edge_index, batch,
           W_rel1, b_rel1, W_root1,
           W_rel2, b_rel2, W_root2,
           W_rel3, b_rel3, W_root3,
           W_lin, b_lin):
    src = edge_index[0].astype(jnp.int32)
    dst = edge_index[1].astype(jnp.int32)
    pad_e = E_PAD - N_EDGES
    pad_src = jnp.arange(pad_e, dtype=jnp.int32) % N_NODES
    pad_dst = N_NODES + (jnp.arange(pad_e, dtype=jnp.int32) % (N_PAD - N_NODES))
    src_p = jnp.concatenate([src, pad_src]).reshape(NW, CHUNKS_PER_TILE, CHUNK)
    dst_p = jnp.concatenate([dst, pad_dst]).reshape(NW, CHUNKS_PER_TILE, CHUNK)
    zrow = jnp.zeros((ROWS_PER_TILE, D), jnp.float32)
    ones_tab = jnp.ones((N_NODES, D), jnp.float32)
    batch2d = batch.astype(jnp.int32).reshape(N_NODES, 1)

    wr1t, wo1t = W_rel1.T, W_root1.T
    wr2t, wo2t = W_rel2.T, W_root2.T
    wr3t, wo3t = W_rel3.T, W_root3.T
    wlt = W_lin.T
    b1 = b_rel1.reshape(1, D)
    b2 = b_rel2.reshape(1, D)
    b3 = b_rel3.reshape(1, D)
    bl = b_lin.reshape(1, D_OUT)

    cnt = _sc_agg(ones_tab, src_p, dst_p, zrow)
    acc1 = _sc_agg(x, src_p, dst_p, zrow)
    h1 = _tc_layer_relu(acc1, cnt, x, wr1t, b1, wo1t)
    acc2 = _sc_agg(h1, src_p, dst_p, zrow)
    h2 = _tc_layer_relu(acc2, cnt, h1, wr2t, b2, wo2t)
    acc3 = _sc_agg(h2, src_p, dst_p, zrow)
    return _tc_pool(acc3, cnt, h2, wr3t, b3, wo3t, batch2d, wlt, bl)


# speedup vs baseline: 27.9520x; 1.0950x over previous
import functools

import jax
import jax.numpy as jnp
from jax import lax
from jax.experimental import pallas as pl
from jax.experimental.pallas import tpu as pltpu
from jax.experimental.pallas import tpu_sc as plsc

N_NODES = 10000
N_EDGES = 320000
N_GRAPHS = 32
D = 128
D_OUT = 64

NC = 2
NS = 16
NW = NC * NS

N_PAD = 10112
ROWS_PER_TILE = N_PAD // NS
CHUNK = 128
E_PAD = 327680
CHUNKS_PER_TILE = E_PAD // (NW * CHUNK)

_mesh = plsc.VectorSubcoreMesh(core_axis_name="c", subcore_axis_name="s",
                               num_cores=NC, num_subcores=NS)


KB = 8
GROUPS = CHUNKS_PER_TILE // KB


def _sc_agg_body(h_hbm, src_hbm, dst_hbm, zrow_hbm, out_hbm,
                 srcA, dstA, srcB, dstB, rows0, rows1, acc, semg, semi):
    c = lax.axis_index("c")
    s = lax.axis_index("s")
    wid = c * NS + s
    row0 = s * ROWS_PER_TILE
    pltpu.sync_copy(zrow_hbm, acc.at[pl.ds(row0, ROWS_PER_TILE)])
    pltpu.sync_copy(src_hbm.at[wid, pl.ds(0, KB)], srcA)
    pltpu.sync_copy(dst_hbm.at[wid, pl.ds(0, KB)], dstA)
    plsc.subcore_barrier()

    rows = (rows0, rows1)

    def process(srcv, dstv):
        pltpu.async_copy(h_hbm.at[srcv.at[0]], rows[0], semg)
        for b in range(KB):
            if b + 1 < KB:
                pltpu.async_copy(h_hbm.at[srcv.at[b + 1]], rows[(b + 1) % 2], semg)
            pltpu.make_async_copy(h_hbm.at[srcv.at[b]], rows[b % 2], semg).wait()
            pltpu.sync_copy(rows[b % 2], acc.at[dstv.at[b]], add=True)

    def stage(off, sv, dv):
        pltpu.async_copy(src_hbm.at[wid, pl.ds(off, KB)], sv, semi)
        pltpu.async_copy(dst_hbm.at[wid, pl.ds(off, KB)], dv, semi)

    def drain(off, sv, dv):
        pltpu.make_async_copy(src_hbm.at[wid, pl.ds(off, KB)], sv, semi).wait()
        pltpu.make_async_copy(dst_hbm.at[wid, pl.ds(off, KB)], dv, semi).wait()

    def step(i, carry):
        off1 = (2 * i + 1) * KB
        stage(off1, srcB, dstB)
        process(srcA, dstA)
        drain(off1, srcB, dstB)
        off2 = jnp.minimum(2 * i + 2, GROUPS - 1) * KB
        stage(off2, srcA, dstA)
        process(srcB, dstB)
        drain(off2, srcA, dstA)
        return carry

    lax.fori_loop(0, GROUPS // 2, step, 0)
    plsc.subcore_barrier()
    pltpu.sync_copy(acc.at[pl.ds(row0, ROWS_PER_TILE)],
                    out_hbm.at[c, pl.ds(row0, ROWS_PER_TILE)])


_sc_agg = pl.kernel(
    _sc_agg_body,
    out_type=jax.ShapeDtypeStruct((NC, N_PAD, D), jnp.float32),
    mesh=_mesh,
    scratch_types=[
        pltpu.VMEM((KB, CHUNK), jnp.int32),
        pltpu.VMEM((KB, CHUNK), jnp.int32),
        pltpu.VMEM((KB, CHUNK), jnp.int32),
        pltpu.VMEM((KB, CHUNK), jnp.int32),
        pltpu.VMEM((CHUNK, D), jnp.float32),
        pltpu.VMEM((CHUNK, D), jnp.float32),
        pltpu.VMEM_SHARED((N_PAD, D), jnp.float32),
        pltpu.SemaphoreType.DMA,
        pltpu.SemaphoreType.DMA,
    ],
)


def _sc_cnt_body(dst_hbm, zrow_hbm, ones_hbm, out_hbm,
                 dstA, dstB, onesv, acc, semi):
    c = lax.axis_index("c")
    s = lax.axis_index("s")
    wid = c * NS + s
    row0 = s * ROWS_PER_TILE
    pltpu.sync_copy(zrow_hbm, acc.at[pl.ds(row0, ROWS_PER_TILE)])
    pltpu.sync_copy(ones_hbm, onesv)
    pltpu.sync_copy(dst_hbm.at[wid, pl.ds(0, KB)], dstA)
    plsc.subcore_barrier()

    def process(dstv):
        for b in range(KB):
            pltpu.sync_copy(onesv, acc.at[dstv.at[b]], add=True)

    def step(i, carry):
        off1 = (2 * i + 1) * KB
        pltpu.async_copy(dst_hbm.at[wid, pl.ds(off1, KB)], dstB, semi)
        process(dstA)
        pltpu.make_async_copy(dst_hbm.at[wid, pl.ds(off1, KB)], dstB, semi).wait()
        off2 = jnp.minimum(2 * i + 2, GROUPS - 1) * KB
        pltpu.async_copy(dst_hbm.at[wid, pl.ds(off2, KB)], dstA, semi)
        process(dstB)
        pltpu.make_async_copy(dst_hbm.at[wid, pl.ds(off2, KB)], dstA, semi).wait()
        return carry

    lax.fori_loop(0, GROUPS // 2, step, 0)
    plsc.subcore_barrier()
    pltpu.sync_copy(acc.at[pl.ds(row0, ROWS_PER_TILE)],
                    out_hbm.at[c, pl.ds(row0, ROWS_PER_TILE)])


_sc_cnt = pl.kernel(
    _sc_cnt_body,
    out_type=jax.ShapeDtypeStruct((NC, N_PAD, D), jnp.float32),
    mesh=_mesh,
    scratch_types=[
        pltpu.VMEM((KB, CHUNK), jnp.int32),
        pltpu.VMEM((KB, CHUNK), jnp.int32),
        pltpu.VMEM((CHUNK, D), jnp.float32),
        pltpu.VMEM_SHARED((N_PAD, D), jnp.float32),
        pltpu.SemaphoreType.DMA,
    ],
)

R = 1000
GRID = N_NODES // R


def _tc_layer_body(relu, acc_ref, cnt_ref, h_ref, wr_ref, b_ref, wo_ref, o_ref):
    ssum = acc_ref[0] + acc_ref[1]
    cnt = cnt_ref[0][:, 0:1] + cnt_ref[1][:, 0:1]
    mean = ssum / jnp.maximum(cnt, 1.0)
    y = (jnp.dot(mean, wr_ref[...], preferred_element_type=jnp.float32)
         + jnp.dot(h_ref[...], wo_ref[...], preferred_element_type=jnp.float32)
         + b_ref[...])
    o_ref[...] = jnp.maximum(y, 0.0) if relu else y


def _make_tc_layer(relu):
    return pl.pallas_call(
        functools.partial(_tc_layer_body, relu),
        grid=(GRID,),
        in_specs=[
            pl.BlockSpec((2, R, D), lambda i: (0, i, 0)),
            pl.BlockSpec((2, R, D), lambda i: (0, i, 0)),
            pl.BlockSpec((R, D), lambda i: (i, 0)),
            pl.BlockSpec((D, D), lambda i: (0, 0)),
            pl.BlockSpec((1, D), lambda i: (0, 0)),
            pl.BlockSpec((D, D), lambda i: (0, 0)),
        ],
        out_specs=pl.BlockSpec((R, D), lambda i: (i, 0)),
        out_shape=jax.ShapeDtypeStruct((N_NODES, D), jnp.float32),
    )


_tc_layer_relu = _make_tc_layer(True)


def _tc_pool_body(acc_ref, cnt_ref, h_ref, wr_ref, b_ref, wo_ref, batch_ref,
                  wl_ref, bl_ref, o_ref, pooled, gcnt):
    i = pl.program_id(0)

    @pl.when(i == 0)
    def _():
        pooled[...] = jnp.zeros_like(pooled)
        gcnt[...] = jnp.zeros_like(gcnt)

    ssum = acc_ref[0] + acc_ref[1]
    cnt = cnt_ref[0][:, 0:1] + cnt_ref[1][:, 0:1]
    mean = ssum / jnp.maximum(cnt, 1.0)
    h3 = (jnp.dot(mean, wr_ref[...], preferred_element_type=jnp.float32)
          + jnp.dot(h_ref[...], wo_ref[...], preferred_element_type=jnp.float32)
          + b_ref[...])
    onehot = (batch_ref[...] ==
              lax.broadcasted_iota(jnp.int32, (R, N_GRAPHS), 1)).astype(jnp.float32)
    pooled[...] += lax.dot_general(onehot, h3, (((0,), (0,)), ((), ())),
                                   preferred_element_type=jnp.float32)
    gcnt[...] += jnp.broadcast_to(jnp.sum(onehot, axis=0)[:, None], (N_GRAPHS, D))

    @pl.when(i == GRID - 1)
    def _():
        pm = pooled[...] / jnp.maximum(gcnt[...], 1.0)
        o_ref[...] = (jnp.dot(pm, wl_ref[...], preferred_element_type=jnp.float32)
                      + bl_ref[...])


_tc_pool = pl.pallas_call(
    _tc_pool_body,
    grid=(GRID,),
    in_specs=[
        pl.BlockSpec((2, R, D), lambda i: (0, i, 0)),
        pl.BlockSpec((2, R, D), lambda i: (0, i, 0)),
        pl.BlockSpec((R, D), lambda i: (i, 0)),
        pl.BlockSpec((D, D), lambda i: (0, 0)),
        pl.BlockSpec((1, D), lambda i: (0, 0)),
        pl.BlockSpec((D, D), lambda i: (0, 0)),
        pl.BlockSpec((R, 1), lambda i: (i, 0)),
        pl.BlockSpec((D, D_OUT), lambda i: (0, 0)),
        pl.BlockSpec((1, D_OUT), lambda i: (0, 0)),
    ],
    out_specs=pl.BlockSpec((N_GRAPHS, D_OUT), lambda i: (0, 0)),
    out_shape=jax.ShapeDtypeStruct((N_GRAPHS, D_OUT), jnp.float32),
    scratch_shapes=[
        pltpu.VMEM((N_GRAPHS, D), jnp.float32),
        pltpu.VMEM((N_GRAPHS, D), jnp.float32),
    ],
)


def kernel(x, edge_index, batch,
           W_rel1, b_rel1, W_root1,
           W_rel2, b_rel2, W_root2,
           W_rel3, b_rel3, W_root3,
           W_lin, b_lin):
    src = edge_index[0].astype(jnp.int32)
    dst = edge_index[1].astype(jnp.int32)
    pad_e = E_PAD - N_EDGES
    pad_src = jnp.arange(pad_e, dtype=jnp.int32) % N_NODES
    pad_dst = N_NODES + (jnp.arange(pad_e, dtype=jnp.int32) % (N_PAD - N_NODES))
    src_p = jnp.concatenate([src, pad_src]).reshape(NW, CHUNKS_PER_TILE, CHUNK)
    dst_p = jnp.concatenate([dst, pad_dst]).reshape(NW, CHUNKS_PER_TILE, CHUNK)
    zrow = jnp.zeros((ROWS_PER_TILE, D), jnp.float32)
    ones_rows = jnp.ones((CHUNK, D), jnp.float32)
    batch2d = batch.astype(jnp.int32).reshape(N_NODES, 1)

    wr1t, wo1t = W_rel1.T, W_root1.T
    wr2t, wo2t = W_rel2.T, W_root2.T
    wr3t, wo3t = W_rel3.T, W_root3.T
    wlt = W_lin.T
    b1 = b_rel1.reshape(1, D)
    b2 = b_rel2.reshape(1, D)
    b3 = b_rel3.reshape(1, D)
    bl = b_lin.reshape(1, D_OUT)

    cnt = _sc_cnt(dst_p, zrow, ones_rows)
    acc1 = _sc_agg(x, src_p, dst_p, zrow)
    h1 = _tc_layer_relu(acc1, cnt, x, wr1t, b1, wo1t)
    acc2 = _sc_agg(h1, src_p, dst_p, zrow)
    h2 = _tc_layer_relu(acc2, cnt, h1, wr2t, b2, wo2t)
    acc3 = _sc_agg(h2, src_p, dst_p, zrow)
    return _tc_pool(acc3, cnt, h2, wr3t, b3, wo3t, batch2d, wlt, bl)
